# Initial kernel scaffold; baseline (speedup 1.0000x reference)
import jax
import jax.numpy as jnp
from jax.experimental import pallas as pl


def kernel(ufeat, ifeat, edge_src, edge_dst, edge_rating, dec_src, dec_dst, W_r, ufc_W, ufc_b, Ps, combine_W):
    raise NotImplementedError("write your pallas kernel here")


# speedup vs baseline: 4.1631x; 4.1631x over previous
import functools
import jax
import jax.numpy as jnp
from jax.experimental import pallas as pl


def _mlp_kernel(x_ref, w1_ref, w2_ref, b_ref, o_ref):
    h = jnp.dot(x_ref[...], w1_ref[...], preferred_element_type=jnp.float32)
    h = jnp.where(h > 0, h, 0.1 * h)
    o_ref[...] = jnp.dot(h, w2_ref[...], preferred_element_type=jnp.float32) + b_ref[0:1, :]


def _dec_kernel(ue_ref, ve_ref, p_ref, cw_ref, o_ref):
    ue = ue_ref[...]
    ve = ve_ref[...]
    s0 = jnp.sum(jnp.dot(ue, p_ref[0], preferred_element_type=jnp.float32) * ve,
                 axis=1, keepdims=True)
    s1 = jnp.sum(jnp.dot(ue, p_ref[1], preferred_element_type=jnp.float32) * ve,
                 axis=1, keepdims=True)
    o_ref[...] = s0 * cw_ref[0:1, :] + s1 * cw_ref[1:2, :]


def kernel(ufeat, ifeat, edge_src, edge_dst, edge_rating, dec_src, dec_dst, W_r, ufc_W, ufc_b, Ps, combine_W):
    n_users = ufeat.shape[0]
    n_items = ifeat.shape[0]
    num_ratings = W_r.shape[0]
    d_in = ufeat.shape[1]
    d_out = ufc_W.shape[1]

    ids_u = edge_src * num_ratings + edge_rating
    ids_i = edge_dst * num_ratings + edge_rating
    nseg_u = n_users * num_ratings
    nseg_i = n_items * num_ratings

    ones = jnp.ones(edge_src.shape, dtype=ufeat.dtype)
    deg_u = jax.ops.segment_sum(ones, ids_u, num_segments=nseg_u)
    deg_i = jax.ops.segment_sum(ones, ids_i, num_segments=nseg_i)
    c_u = jnp.where(deg_u > 0, jax.lax.rsqrt(jnp.maximum(deg_u, 1.0)), 0.0)
    c_i = jnp.where(deg_i > 0, jax.lax.rsqrt(jnp.maximum(deg_i, 1.0)), 0.0)

    msg_ui = ufeat[edge_src] * c_u[ids_u][:, None]
    item_acc = jax.ops.segment_sum(msg_ui, ids_i, num_segments=nseg_i)
    msg_iu = ifeat[edge_dst] * c_i[ids_i][:, None]
    user_acc = jax.ops.segment_sum(msg_iu, ids_u, num_segments=nseg_u)

    item_x = (item_acc * c_i[:, None]).reshape(n_items, num_ratings * d_in)
    user_x = (user_acc * c_u[:, None]).reshape(n_users, num_ratings * d_in)

    w1 = W_r.reshape(num_ratings * d_in, W_r.shape[2])
    b8 = jnp.broadcast_to(ufc_b[None, :], (8, d_out))

    x_all = jnp.concatenate([user_x, item_x], axis=0)
    n_all = x_all.shape[0]
    bn = 2000
    grid = n_all // bn
    out_all = pl.pallas_call(
        _mlp_kernel,
        grid=(grid,),
        in_specs=[
            pl.BlockSpec((bn, num_ratings * d_in), lambda i: (i, 0)),
            pl.BlockSpec(w1.shape, lambda i: (0, 0)),
            pl.BlockSpec(ufc_W.shape, lambda i: (0, 0)),
            pl.BlockSpec((8, d_out), lambda i: (0, 0)),
        ],
        out_specs=pl.BlockSpec((bn, d_out), lambda i: (i, 0)),
        out_shape=jax.ShapeDtypeStruct((n_all, d_out), jnp.float32),
    )(x_all, w1, ufc_W, b8)
    user_out = out_all[:n_users]
    item_out = out_all[n_users:]

    ue = user_out[dec_src]
    ve = item_out[dec_dst]
    n_dec = ue.shape[0]
    n_classes = combine_W.shape[1]
    cw_pad = jnp.zeros((combine_W.shape[0], 8), dtype=combine_W.dtype).at[:, :n_classes].set(combine_W)
    be = 2000
    dgrid = n_dec // be
    pred_pad = pl.pallas_call(
        _dec_kernel,
        grid=(dgrid,),
        in_specs=[
            pl.BlockSpec((be, d_out), lambda i: (i, 0)),
            pl.BlockSpec((be, d_out), lambda i: (i, 0)),
            pl.BlockSpec(Ps.shape, lambda i: (0, 0, 0)),
            pl.BlockSpec((combine_W.shape[0], 8), lambda i: (0, 0)),
        ],
        out_specs=pl.BlockSpec((be, 8), lambda i: (i, 0)),
        out_shape=jax.ShapeDtypeStruct((n_dec, 8), jnp.float32),
    )(ue, ve, Ps, cw_pad)
    return pred_pad[:, :n_classes]
